# Initial kernel scaffold; baseline (speedup 1.0000x reference)
import jax
import jax.numpy as jnp
from jax.experimental import pallas as pl


def kernel(encoder_hiddens, dig_users, responder, Wih_s, Whh_s, bih_s, bhh_s, Wih_a, Whh_a, bih_a, bhh_a, Wih_o, Whh_o, bih_o, bhh_o, fc1_W, fc1_b, fc2_W, fc2_b):
    raise NotImplementedError("write your pallas kernel here")


# speedup vs baseline: 10.7992x; 10.7992x over previous
import jax
import jax.numpy as jnp
from jax.experimental import pallas as pl
from jax.experimental.pallas import tpu as pltpu

B = 256
W = 20
H2 = 512
R = 10
D = 256


def _gru_tail(gi, gh, h):
    i_r = gi[..., :D]
    i_z = gi[..., D:2 * D]
    i_n = gi[..., 2 * D:]
    h_r = gh[..., :D]
    h_z = gh[..., D:2 * D]
    h_n = gh[..., 2 * D:]
    r = jax.nn.sigmoid(i_r + h_r)
    z = jax.nn.sigmoid(i_z + h_z)
    n = jnp.tanh(i_n + r * h_n)
    return (1.0 - z) * n + z * h


def _adr_kernel(enc_ref, ohs_ref, oha_ref, ohr_ref, sel_ref,
                Wcat_eh_ref, bcat_eh_ref, Whh_o_ref, bhh_o_ref,
                Wcat_s_ref, bcat_s_ref, Wcat_a_ref, bcat_a_ref,
                W1_ref, V_ref, fc1_b_ref, fc2_W_ref, fc2_b_ref,
                out_ref, A_ref):
    bf16 = jnp.bfloat16
    f32 = jnp.float32
    T = pl.program_id(0)

    @pl.when(T == 0)
    def _init():
        A_ref[...] = jnp.zeros((R * B, D), f32)

    eh = enc_ref[0]
    ohs = ohs_ref[0]
    oha = oha_ref[0]
    A2 = A_ref[...]
    A3 = A2.reshape(R, B, D)
    spk_v = jnp.sum(ohs[:, :, None] * A3, axis=0)
    adr_v = jnp.sum(oha[:, :, None] * A3, axis=0)

    GI = jnp.dot(eh, Wcat_eh_ref[...],
                 preferred_element_type=f32) + bcat_eh_ref[...]
    GHO = jnp.dot(A2.astype(bf16), Whh_o_ref[...],
                  preferred_element_type=f32) + bhh_o_ref[...]
    Sc = jnp.dot(spk_v.astype(bf16), Wcat_s_ref[...],
                 preferred_element_type=f32) + bcat_s_ref[...]
    Ac = jnp.dot(adr_v.astype(bf16), Wcat_a_ref[...],
                 preferred_element_type=f32) + bcat_a_ref[...]

    gio = GI[:, :3 * D][None]
    GHO3 = GHO.reshape(R, B, 3 * D)
    new_o = _gru_tail(gio, GHO3, A3)

    gis = GI[:, 3 * D:6 * D] + Ac[:, :3 * D]
    new_s = _gru_tail(gis, Sc[:, :3 * D], spk_v)
    gia = GI[:, 6 * D:] + Sc[:, 3 * D:]
    new_a = _gru_tail(gia, Ac[:, 3 * D:], adr_v)

    mo = (1.0 - ohs - oha)[:, :, None]
    A_new = (mo * new_o + ohs[:, :, None] * new_s[None]
             + oha[:, :, None] * new_a[None]).reshape(R * B, D)
    A_ref[...] = A_new

    @pl.when(T == W - 1)
    def _head():
        A2h = A_new
        A3h = A2h.reshape(R, B, D)

        ohr = ohr_ref[...]
        A_res = jnp.sum(ohr[:, :, None] * A3h, axis=0)

        m = jnp.max(A2h.reshape(R * B, D // 8, 8), axis=-1)
        m3 = m.reshape(R, B, D // 8)
        cc = jnp.dot(A_res, W1_ref[...],
                     preferred_element_type=f32) + fc1_b_ref[...]
        for j in range(R - 2):
            u_j = jnp.maximum(jnp.maximum(m3[j], m3[j + 1]), m3[j + 2])
            cc = cc + jnp.dot(u_j, V_ref[j], preferred_element_type=f32)
        cc = jnp.tanh(cc)

        q = jnp.sum(A3h * cc[None], axis=-1)
        lo = sel_ref[...]
        o9 = q[:R - 1] * lo + q[1:] * (1.0 - lo)
        out = jax.lax.dot_general(
            o9, fc2_W_ref[...], (((0,), (0,)), ((), ())),
            preferred_element_type=f32) + fc2_b_ref[...]
        out_ref[...] = out


def kernel(encoder_hiddens, dig_users, responder,
           Wih_s, Whh_s, bih_s, bhh_s,
           Wih_a, Whh_a, bih_a, bhh_a,
           Wih_o, Whh_o, bih_o, bhh_o,
           fc1_W, fc1_b, fc2_W, fc2_b):
    f32 = jnp.float32
    bf16 = jnp.bfloat16
    spk = dig_users[..., 0]
    adr = dig_users[..., 1]
    rids = jnp.arange(R, dtype=jnp.int32)[None, :, None]
    ohs = (spk.T[:, None, :] == rids).astype(f32)
    oha = (adr.T[:, None, :] == rids).astype(f32)
    ohr = (jnp.arange(R, dtype=jnp.int32)[:, None]
           == responder[None, :]).astype(f32)
    sel = (jnp.arange(R - 1, dtype=jnp.int32)[:, None]
           < responder[None, :]).astype(f32)
    enc = jnp.transpose(encoder_hiddens, (1, 0, 2)).astype(bf16)

    Wcat_eh = jnp.concatenate(
        [Wih_o, Wih_s[:H2], Wih_a[:H2]], axis=1).astype(bf16)
    bcat_eh = jnp.concatenate([bih_o, bih_s, bih_a])[None]
    Wcat_s = jnp.concatenate([Whh_s, Wih_a[H2:]], axis=1).astype(bf16)
    bcat_s = jnp.concatenate([bhh_s, jnp.zeros_like(bhh_a)])[None]
    Wcat_a = jnp.concatenate([Wih_s[H2:], Whh_a], axis=1).astype(bf16)
    bcat_a = jnp.concatenate([jnp.zeros_like(bhh_s), bhh_a])[None]
    V = jnp.stack([fc1_W[D:][j::8] for j in range(8)])
    W1 = fc1_W[:D]

    def _full(shape):
        nd = len(shape)
        return pl.BlockSpec(shape, lambda t: (0,) * nd)

    out, A_flat = pl.pallas_call(
        _adr_kernel,
        grid=(W,),
        in_specs=[
            pl.BlockSpec((1, B, H2), lambda t: (t, 0, 0)),
            pl.BlockSpec((1, R, B), lambda t: (t, 0, 0)),
            pl.BlockSpec((1, R, B), lambda t: (t, 0, 0)),
            _full((R, B)), _full((R - 1, B)),
            _full((H2, 9 * D)), _full((1, 9 * D)),
            _full((D, 3 * D)), _full((1, 3 * D)),
            _full((D, 6 * D)), _full((1, 6 * D)),
            _full((D, 6 * D)), _full((1, 6 * D)),
            _full((D, D)), _full((8, D // 8, D)), _full((1, D)),
            _full((R - 1, R - 1)), _full((1, R - 1)),
        ],
        out_specs=(
            _full((B, R - 1)),
            _full((R * B, D)),
        ),
        out_shape=(
            jax.ShapeDtypeStruct((B, R - 1), f32),
            jax.ShapeDtypeStruct((R * B, D), f32),
        ),
        compiler_params=pltpu.CompilerParams(
            dimension_semantics=("arbitrary",),
            vmem_limit_bytes=100 * 1024 * 1024,
        ),
    )(enc, ohs, oha, ohr, sel,
      Wcat_eh, bcat_eh, Whh_o.astype(bf16), bhh_o[None],
      Wcat_s, bcat_s, Wcat_a, bcat_a,
      W1, V, fc1_b[None], fc2_W, fc2_b[None])

    A = jnp.transpose(A_flat.reshape(R, B, D), (1, 0, 2))
    return (out, A)
